# baseline (speedup 1.0000x reference)
import jax
import jax.numpy as jnp
from jax.experimental.layout import Format, Layout, with_layout_constraint
from jax.experimental import pallas as pl
from jax.experimental.pallas import tpu as pltpu
from jax.experimental.pallas import tpu_sc as plsc

T = 2048
C = 768
V = 50257
BV = 1024
NBLK = (V + BV - 1) // BV
NCHUNK = 4
BPC = (NBLK + NCHUNK - 1) // NCHUNK


def _sc_gather(tok_embds, ids):
    mesh = plsc.VectorSubcoreMesh(core_axis_name="c", subcore_axis_name="s")
    nw = 32
    b_per_w = T // nw

    @pl.kernel(
        out_type=jax.ShapeDtypeStruct((T, C), tok_embds.dtype),
        mesh=mesh,
        scratch_types=[
            pltpu.VMEM((b_per_w,), jnp.int32),
            pltpu.VMEM((b_per_w, C), jnp.float32),
            pltpu.SemaphoreType.DMA,
        ],
    )
    def gather_kernel(table_hbm, idx_hbm, out_hbm, idx_v, rows_v, sem):
        wid = jax.lax.axis_index("s") * 2 + jax.lax.axis_index("c")
        base = wid * b_per_w
        pltpu.sync_copy(idx_hbm.at[pl.ds(base, b_per_w)], idx_v)
        pltpu.async_copy(table_hbm.at[idx_v], rows_v, sem).wait()
        pltpu.sync_copy(rows_v, out_hbm.at[pl.ds(base, b_per_w)])

    return gather_kernel(tok_embds, ids)


def _mm_body(flag_ref, x_ref, p_ref, wt_ref, b_ref, o_ref, xbt_ref):
    j = pl.program_id(0)

    @pl.when(j == 0)
    def _():
        xbt_ref[...] = jnp.transpose((x_ref[...] + p_ref[...]).astype(jnp.bfloat16))

    acc = jnp.dot(
        wt_ref[...].astype(jnp.bfloat16),
        xbt_ref[...],
        preferred_element_type=jnp.float32,
    )

    @pl.when(flag_ref[0] == 0)
    def _():
        o_ref[...] = acc.reshape(BV, 1, T)

    @pl.when(flag_ref[0] != 0)
    def _():
        o_ref[...] = (acc + jnp.transpose(b_ref[...])).reshape(BV, 1, T)


def _mm_chunk(bflag, x, pos, wt, b2, block_off, nblk, vc):
    grid_spec = pltpu.PrefetchScalarGridSpec(
        num_scalar_prefetch=1,
        grid=(nblk,),
        in_specs=[
            pl.BlockSpec((T, C), lambda j, f: (0, 0)),
            pl.BlockSpec((T, C), lambda j, f: (0, 0)),
            pl.BlockSpec((BV, C), lambda j, f: (block_off + j, 0)),
            pl.BlockSpec((1, BV), lambda j, f: (0, block_off + j)),
        ],
        out_specs=pl.BlockSpec((BV, 1, T), lambda j, f: (j, 0, 0)),
        scratch_shapes=[pltpu.VMEM((C, T), jnp.bfloat16)],
    )
    return pl.pallas_call(
        _mm_body,
        grid_spec=grid_spec,
        out_shape=jax.ShapeDtypeStruct((vc, 1, T), jnp.float32),
        compiler_params=pltpu.CompilerParams(dimension_semantics=("arbitrary",)),
    )(bflag, x, pos, wt, b2)


def kernel(inputs, tok_embds, pos_embds, lm_head_w, lm_head_b):
    B, Tin = inputs.shape
    ids = inputs.reshape(B * Tin).astype(jnp.int32)
    x = _sc_gather(tok_embds, ids)
    wt = lm_head_w.T
    b2 = lm_head_b.reshape(1, V)
    bflag = jnp.any(lm_head_b != 0).astype(jnp.int32).reshape(1)
    out3 = _mm_chunk(bflag, x, pos_embds, wt, b2, 0, NBLK, V)
    return jnp.transpose(out3, (1, 2, 0))


# speedup vs baseline: 1.0306x; 1.0306x over previous
import jax
import jax.numpy as jnp
from jax.experimental.layout import Format, Layout, with_layout_constraint
from jax.experimental import pallas as pl
from jax.experimental.pallas import tpu as pltpu
from jax.experimental.pallas import tpu_sc as plsc

T = 2048
C = 768
V = 50257
BV = 768
NBLK = (V + BV - 1) // BV
NCHUNK = 4
BPC = (NBLK + NCHUNK - 1) // NCHUNK


def _sc_gather(tok_embds, ids):
    mesh = plsc.VectorSubcoreMesh(core_axis_name="c", subcore_axis_name="s")
    nw = 32
    b_per_w = T // nw

    @pl.kernel(
        out_type=jax.ShapeDtypeStruct((T, C), tok_embds.dtype),
        mesh=mesh,
        scratch_types=[
            pltpu.VMEM((b_per_w,), jnp.int32),
            pltpu.VMEM((b_per_w, C), jnp.float32),
            pltpu.SemaphoreType.DMA,
        ],
    )
    def gather_kernel(table_hbm, idx_hbm, out_hbm, idx_v, rows_v, sem):
        wid = jax.lax.axis_index("s") * 2 + jax.lax.axis_index("c")
        base = wid * b_per_w
        pltpu.sync_copy(idx_hbm.at[pl.ds(base, b_per_w)], idx_v)
        pltpu.async_copy(table_hbm.at[idx_v], rows_v, sem).wait()
        pltpu.sync_copy(rows_v, out_hbm.at[pl.ds(base, b_per_w)])

    return gather_kernel(tok_embds, ids)


def _mm_body(x_ref, p_ref, wt_ref, b_ref, o_ref, xbt_ref):
    j = pl.program_id(0)

    @pl.when(j == 0)
    def _():
        xbt_ref[...] = jnp.transpose((x_ref[...] + p_ref[...]).astype(jnp.bfloat16))

    acc = jnp.dot(
        wt_ref[...].astype(jnp.bfloat16),
        xbt_ref[...],
        preferred_element_type=jnp.float32,
    )
    o_ref[...] = (acc + jnp.transpose(b_ref[...])).reshape(BV, 1, T)


def _mm_chunk(x, pos, wt, b2, block_off, nblk, vc):
    return pl.pallas_call(
        _mm_body,
        grid=(nblk,),
        in_specs=[
            pl.BlockSpec((T, C), lambda j: (0, 0)),
            pl.BlockSpec((T, C), lambda j: (0, 0)),
            pl.BlockSpec((BV, C), lambda j: (block_off + j, 0)),
            pl.BlockSpec((1, BV), lambda j: (0, block_off + j)),
        ],
        out_specs=pl.BlockSpec((BV, 1, T), lambda j: (j, 0, 0)),
        out_shape=jax.ShapeDtypeStruct((vc, 1, T), jnp.float32),
        scratch_shapes=[pltpu.VMEM((C, T), jnp.bfloat16)],
        compiler_params=pltpu.CompilerParams(dimension_semantics=("arbitrary",)),
    )(x, pos, wt, b2)


def kernel(inputs, tok_embds, pos_embds, lm_head_w, lm_head_b):
    B, Tin = inputs.shape
    ids = inputs.reshape(B * Tin).astype(jnp.int32)
    x = _sc_gather(tok_embds, ids)
    wt = lm_head_w.T
    b2 = lm_head_b.reshape(1, V)
    out3 = _mm_chunk(x, pos_embds, wt, b2, 0, NBLK, V)
    return jnp.transpose(out3, (1, 2, 0))


# speedup vs baseline: 1.0335x; 1.0029x over previous
import jax
import jax.numpy as jnp
from jax.experimental.layout import Format, Layout, with_layout_constraint
from jax.experimental import pallas as pl
from jax.experimental.pallas import tpu as pltpu
from jax.experimental.pallas import tpu_sc as plsc

T = 2048
C = 768
V = 50257
BV = 1024
NBLK = (V + BV - 1) // BV
NCHUNK = 4
BPC = (NBLK + NCHUNK - 1) // NCHUNK


def _sc_gather(tok_embds, ids):
    mesh = plsc.VectorSubcoreMesh(core_axis_name="c", subcore_axis_name="s")
    nw = 32
    b_per_w = T // nw

    @pl.kernel(
        out_type=jax.ShapeDtypeStruct((T, C), tok_embds.dtype),
        mesh=mesh,
        scratch_types=[
            pltpu.VMEM((b_per_w,), jnp.int32),
            pltpu.VMEM((b_per_w, C), jnp.float32),
            pltpu.SemaphoreType.DMA,
        ],
    )
    def gather_kernel(table_hbm, idx_hbm, out_hbm, idx_v, rows_v, sem):
        wid = jax.lax.axis_index("s") * 2 + jax.lax.axis_index("c")
        base = wid * b_per_w
        pltpu.sync_copy(idx_hbm.at[pl.ds(base, b_per_w)], idx_v)
        pltpu.async_copy(table_hbm.at[idx_v], rows_v, sem).wait()
        pltpu.sync_copy(rows_v, out_hbm.at[pl.ds(base, b_per_w)])

    return gather_kernel(tok_embds, ids)


def _mm_body(x_ref, p_ref, wt_ref, b_ref, o_ref, xbt_ref):
    j = pl.program_id(0)

    @pl.when(j == 0)
    def _():
        xbt_ref[...] = jnp.transpose((x_ref[...] + p_ref[...]).astype(jnp.bfloat16))

    acc = jnp.dot(
        wt_ref[...].astype(jnp.bfloat16),
        xbt_ref[...],
        preferred_element_type=jnp.float32,
    )
    o_ref[...] = (acc + jnp.transpose(b_ref[...])).reshape(BV, 1, T)


def _mm_chunk(x, pos, wt, b2, block_off, nblk, vc):
    return pl.pallas_call(
        _mm_body,
        grid=(nblk,),
        in_specs=[
            pl.BlockSpec((T, C), lambda j: (0, 0)),
            pl.BlockSpec((T, C), lambda j: (0, 0)),
            pl.BlockSpec((BV, C), lambda j: (block_off + j, 0)),
            pl.BlockSpec((1, BV), lambda j: (0, block_off + j)),
        ],
        out_specs=pl.BlockSpec((BV, 1, T), lambda j: (j, 0, 0)),
        out_shape=jax.ShapeDtypeStruct((vc, 1, T), jnp.float32),
        scratch_shapes=[pltpu.VMEM((C, T), jnp.bfloat16)],
        compiler_params=pltpu.CompilerParams(dimension_semantics=("arbitrary",)),
    )(x, pos, wt, b2)


def kernel(inputs, tok_embds, pos_embds, lm_head_w, lm_head_b):
    B, Tin = inputs.shape
    ids = inputs.reshape(B * Tin).astype(jnp.int32)
    x = _sc_gather(tok_embds, ids)
    wt = lm_head_w.T
    b2 = lm_head_b.reshape(1, V)
    out3 = _mm_chunk(x, pos_embds, wt, b2, 0, NBLK, V)
    return jnp.transpose(out3, (1, 2, 0))


# speedup vs baseline: 1.0344x; 1.0008x over previous
import jax
import jax.numpy as jnp
from jax.experimental import pallas as pl
from jax.experimental.pallas import tpu as pltpu
from jax.experimental.pallas import tpu_sc as plsc

T = 2048
C = 768
V = 50257
BV = 1024
NBLK = (V + BV - 1) // BV


def _sc_gather(tok_embds, ids):
    mesh = plsc.VectorSubcoreMesh(core_axis_name="c", subcore_axis_name="s")
    nw = 32
    b_per_w = T // nw

    @pl.kernel(
        out_type=jax.ShapeDtypeStruct((T, C), tok_embds.dtype),
        mesh=mesh,
        scratch_types=[
            pltpu.VMEM((b_per_w,), jnp.int32),
            pltpu.VMEM((b_per_w, C), jnp.float32),
            pltpu.SemaphoreType.DMA,
        ],
    )
    def gather_kernel(table_hbm, idx_hbm, out_hbm, idx_v, rows_v, sem):
        wid = jax.lax.axis_index("s") * 2 + jax.lax.axis_index("c")
        base = wid * b_per_w
        pltpu.sync_copy(idx_hbm.at[pl.ds(base, b_per_w)], idx_v)
        pltpu.async_copy(table_hbm.at[idx_v], rows_v, sem).wait()
        pltpu.sync_copy(rows_v, out_hbm.at[pl.ds(base, b_per_w)])

    return gather_kernel(tok_embds, ids)


def _mm_body(x_ref, p_ref, wt_ref, b_ref, o_ref, xbt_ref):
    j = pl.program_id(0)

    @pl.when(j == 0)
    def _():
        xbt_ref[...] = jnp.transpose((x_ref[...] + p_ref[...]).astype(jnp.bfloat16))

    acc = jnp.dot(
        wt_ref[...].astype(jnp.bfloat16),
        xbt_ref[...],
        preferred_element_type=jnp.float32,
    )
    o_ref[...] = (acc + jnp.transpose(b_ref[...])).reshape(BV, 1, T)


def _mm_transposed(x, pos, wt, b2):
    return pl.pallas_call(
        _mm_body,
        grid=(NBLK,),
        in_specs=[
            pl.BlockSpec((T, C), lambda j: (0, 0)),
            pl.BlockSpec((T, C), lambda j: (0, 0)),
            pl.BlockSpec((BV, C), lambda j: (j, 0)),
            pl.BlockSpec((1, BV), lambda j: (0, j)),
        ],
        out_specs=pl.BlockSpec((BV, 1, T), lambda j: (j, 0, 0)),
        out_shape=jax.ShapeDtypeStruct((V, 1, T), jnp.float32),
        scratch_shapes=[pltpu.VMEM((C, T), jnp.bfloat16)],
        compiler_params=pltpu.CompilerParams(dimension_semantics=("arbitrary",)),
    )(x, pos, wt, b2)


def kernel(inputs, tok_embds, pos_embds, lm_head_w, lm_head_b):
    B, Tin = inputs.shape
    ids = inputs.reshape(B * Tin).astype(jnp.int32)
    x = _sc_gather(tok_embds, ids)
    out3 = _mm_transposed(x, pos_embds, lm_head_w.T, lm_head_b.reshape(1, V))
    return jnp.transpose(out3, (1, 2, 0))
